# Initial kernel scaffold; baseline (speedup 1.0000x reference)
import jax
import jax.numpy as jnp
from jax.experimental import pallas as pl


def kernel(x, lengths, emb_table, W1, b1, W2, b2):
    raise NotImplementedError("write your pallas kernel here")


# speedup vs baseline: 8.0726x; 8.0726x over previous
import functools

import jax
import jax.numpy as jnp
from jax import lax
from jax.experimental import pallas as pl
from jax.experimental.pallas import tpu as pltpu
from jax.experimental.pallas import tpu_sc as plsc

B, L, V, D = 4096, 200, 100000, 64
NC, NS = 2, 16
NW = NC * NS
SPB = B // NW
LH = L // 2


def _pool_sc(x2, lengths, emb_table):
    mesh = plsc.VectorSubcoreMesh(core_axis_name="c", subcore_axis_name="s")

    @functools.partial(
        pl.kernel,
        out_type=(
            jax.ShapeDtypeStruct((B, D), jnp.float32),
            jax.ShapeDtypeStruct((B, D), jnp.float32),
        ),
        mesh=mesh,
        compiler_params=pltpu.CompilerParams(use_tc_tiling_on_sc=False),
        scratch_types=[
            pltpu.VMEM((SPB, 2, LH), jnp.int32),
            pltpu.VMEM((SPB + 16,), jnp.int32),
            pltpu.VMEM((L, D), jnp.float32),
            pltpu.VMEM((SPB, D), jnp.float32),
            pltpu.VMEM((SPB, D), jnp.float32),
            pltpu.SemaphoreType.DMA,
        ],
    )
    def pool(x_hbm, len_hbm, table_hbm, sum_hbm, max_hbm,
             idx_v, len_v, rows_v, sum_v, max_v, sem):
        wid = lax.axis_index("s") * NC + lax.axis_index("c")
        base = wid * SPB
        pltpu.sync_copy(x_hbm.at[pl.ds(base, SPB)], idx_v)
        pltpu.sync_copy(len_hbm.at[pl.ds(base, SPB)], len_v.at[pl.ds(0, SPB)])

        neg = jnp.finfo(jnp.float32).min

        def sample_body(b, _):
            c1 = pltpu.async_copy(table_hbm.at[idx_v.at[b, 0]],
                                  rows_v.at[pl.ds(0, LH)], sem)
            c2 = pltpu.async_copy(table_hbm.at[idx_v.at[b, 1]],
                                  rows_v.at[pl.ds(LH, LH)], sem)
            c1.wait()
            c2.wait()

            n = len_v[pl.ds(b, 16)][0]
            init = (jnp.zeros((16,), jnp.float32),) * 4 + \
                   (jnp.full((16,), neg, jnp.float32),) * 4

            def ph1(r, c):
                s0, s1, s2, s3, m0, m1, m2, m3 = c
                v0 = rows_v[r, pl.ds(0, 16)]
                v1 = rows_v[r, pl.ds(16, 16)]
                v2 = rows_v[r, pl.ds(32, 16)]
                v3 = rows_v[r, pl.ds(48, 16)]
                return (s0 + v0, s1 + v1, s2 + v2, s3 + v3,
                        jnp.maximum(m0, v0), jnp.maximum(m1, v1),
                        jnp.maximum(m2, v2), jnp.maximum(m3, v3))

            def ph2(r, c):
                s0, s1, s2, s3, m0, m1, m2, m3 = c
                v0 = rows_v[r, pl.ds(0, 16)]
                v1 = rows_v[r, pl.ds(16, 16)]
                v2 = rows_v[r, pl.ds(32, 16)]
                v3 = rows_v[r, pl.ds(48, 16)]
                return (s0 + v0, s1 + v1, s2 + v2, s3 + v3, m0, m1, m2, m3)

            c = lax.fori_loop(0, n, ph1, init)
            s0, s1, s2, s3, m0, m1, m2, m3 = lax.fori_loop(n, L, ph2, c)
            sum_v[b, pl.ds(0, 16)] = s0
            sum_v[b, pl.ds(16, 16)] = s1
            sum_v[b, pl.ds(32, 16)] = s2
            sum_v[b, pl.ds(48, 16)] = s3
            max_v[b, pl.ds(0, 16)] = m0
            max_v[b, pl.ds(16, 16)] = m1
            max_v[b, pl.ds(32, 16)] = m2
            max_v[b, pl.ds(48, 16)] = m3
            return 0

        lax.fori_loop(0, SPB, sample_body, 0)

        pltpu.sync_copy(sum_v, sum_hbm.at[pl.ds(base, SPB)])
        pltpu.sync_copy(max_v, max_hbm.at[pl.ds(base, SPB)])

    return pool(x2, lengths, emb_table)


def _mlp_kernel(sum_ref, max_ref, len_ref, w1a_ref, w1b_ref, b1_ref,
                w2_ref, b2_ref, out_ref):
    mean = sum_ref[...] / len_ref[...]
    h = jnp.dot(mean, w1a_ref[...], preferred_element_type=jnp.float32)
    h += jnp.dot(max_ref[...], w1b_ref[...], preferred_element_type=jnp.float32)
    h = jnp.maximum(h + b1_ref[...], 0.0)
    out = jnp.dot(h, w2_ref[...], preferred_element_type=jnp.float32)
    out_ref[...] = out + b2_ref[...]


def _mlp_tc(sum_pooled, max_pooled, lengths_f, W1, b1, W2, b2):
    H = W1.shape[1]
    C = W2.shape[1]
    W1a = W1[:D]
    W1b = W1[D:]
    BLK = 1024
    grid = B // BLK
    return pl.pallas_call(
        _mlp_kernel,
        grid=(grid,),
        in_specs=[
            pl.BlockSpec((BLK, D), lambda i: (i, 0)),
            pl.BlockSpec((BLK, D), lambda i: (i, 0)),
            pl.BlockSpec((BLK, 1), lambda i: (i, 0)),
            pl.BlockSpec((D, H), lambda i: (0, 0)),
            pl.BlockSpec((D, H), lambda i: (0, 0)),
            pl.BlockSpec((1, H), lambda i: (0, 0)),
            pl.BlockSpec((H, C), lambda i: (0, 0)),
            pl.BlockSpec((1, C), lambda i: (0, 0)),
        ],
        out_specs=pl.BlockSpec((BLK, C), lambda i: (i, 0)),
        out_shape=jax.ShapeDtypeStruct((B, C), jnp.float32),
    )(sum_pooled, max_pooled, lengths_f, W1a, W1b, b1[None, :], W2, b2[None, :])


def kernel(x, lengths, emb_table, W1, b1, W2, b2):
    x2 = x.reshape(B, 2, LH).astype(jnp.int32)
    lengths = lengths.astype(jnp.int32)
    sum_pooled, max_pooled = _pool_sc(x2, lengths, emb_table)
    lengths_f = lengths.astype(jnp.float32)[:, None]
    return _mlp_tc(sum_pooled, max_pooled, lengths_f, W1, b1, W2, b2)


# speedup vs baseline: 12.0339x; 1.4907x over previous
import functools

import jax
import jax.numpy as jnp
from jax import lax
from jax.experimental import pallas as pl
from jax.experimental.pallas import tpu as pltpu
from jax.experimental.pallas import tpu_sc as plsc

B, L, V, D = 4096, 200, 100000, 64
NC, NS = 2, 16
NW = NC * NS
SPB = B // NW
LH = L // 2


def _pool_sc(x2, lengths, emb_table):
    mesh = plsc.VectorSubcoreMesh(core_axis_name="c", subcore_axis_name="s")

    @functools.partial(
        pl.kernel,
        out_type=(
            jax.ShapeDtypeStruct((B, D), jnp.float32),
            jax.ShapeDtypeStruct((B, D), jnp.float32),
        ),
        mesh=mesh,
        compiler_params=pltpu.CompilerParams(use_tc_tiling_on_sc=False),
        scratch_types=[
            pltpu.VMEM((SPB, 2, LH), jnp.int32),
            pltpu.VMEM((SPB + 16,), jnp.int32),
            pltpu.VMEM((L, D), jnp.float32),
            pltpu.VMEM((L, D), jnp.float32),
            pltpu.VMEM((SPB, D), jnp.float32),
            pltpu.VMEM((SPB, D), jnp.float32),
            pltpu.SemaphoreType.DMA,
            pltpu.SemaphoreType.DMA,
        ],
    )
    def pool(x_hbm, len_hbm, table_hbm, sum_hbm, max_hbm,
             idx_v, len_v, rows0_v, rows1_v, sum_v, max_v, sem0, sem1):
        wid = lax.axis_index("s") * NC + lax.axis_index("c")
        base = wid * SPB
        pltpu.sync_copy(x_hbm.at[pl.ds(base, SPB)], idx_v)
        pltpu.sync_copy(len_hbm.at[pl.ds(base, SPB)], len_v.at[pl.ds(0, SPB)])

        neg = jnp.finfo(jnp.float32).min

        def start_gather(b, buf, sem):
            pltpu.async_copy(table_hbm.at[idx_v.at[b, 0]],
                             buf.at[pl.ds(0, LH)], sem)
            pltpu.async_copy(table_hbm.at[idx_v.at[b, 1]],
                             buf.at[pl.ds(LH, LH)], sem)

        def wait_gather(buf, sem):
            pltpu.make_async_copy(table_hbm.at[pl.ds(0, L)], buf, sem).wait()

        def compute(b, rows_v):
            n = len_v[pl.ds(b, 16)][0]
            init = (jnp.zeros((16,), jnp.float32),) * 4 + \
                   (jnp.full((16,), neg, jnp.float32),) * 4

            def ph1(r, c):
                s0, s1, s2, s3, m0, m1, m2, m3 = c
                v0 = rows_v[r, pl.ds(0, 16)]
                v1 = rows_v[r, pl.ds(16, 16)]
                v2 = rows_v[r, pl.ds(32, 16)]
                v3 = rows_v[r, pl.ds(48, 16)]
                return (s0 + v0, s1 + v1, s2 + v2, s3 + v3,
                        jnp.maximum(m0, v0), jnp.maximum(m1, v1),
                        jnp.maximum(m2, v2), jnp.maximum(m3, v3))

            def ph2(r, c):
                s0, s1, s2, s3, m0, m1, m2, m3 = c
                v0 = rows_v[r, pl.ds(0, 16)]
                v1 = rows_v[r, pl.ds(16, 16)]
                v2 = rows_v[r, pl.ds(32, 16)]
                v3 = rows_v[r, pl.ds(48, 16)]
                return (s0 + v0, s1 + v1, s2 + v2, s3 + v3, m0, m1, m2, m3)

            c = lax.fori_loop(0, n, ph1, init)
            s0, s1, s2, s3, m0, m1, m2, m3 = lax.fori_loop(n, L, ph2, c)
            sum_v[b, pl.ds(0, 16)] = s0
            sum_v[b, pl.ds(16, 16)] = s1
            sum_v[b, pl.ds(32, 16)] = s2
            sum_v[b, pl.ds(48, 16)] = s3
            max_v[b, pl.ds(0, 16)] = m0
            max_v[b, pl.ds(16, 16)] = m1
            max_v[b, pl.ds(32, 16)] = m2
            max_v[b, pl.ds(48, 16)] = m3

        bufs = ((rows0_v, sem0), (rows1_v, sem1))
        start_gather(0, rows0_v, sem0)

        @pl.loop(0, SPB, step=2)
        def _pair(i):
            for k in range(2):
                b = i + k
                buf, sem = bufs[k]
                nbuf, nsem = bufs[1 - k]

                @pl.when(b + 1 < SPB)
                def _():
                    start_gather(b + 1, nbuf, nsem)

                wait_gather(buf, sem)
                compute(b, buf)

        pltpu.sync_copy(sum_v, sum_hbm.at[pl.ds(base, SPB)])
        pltpu.sync_copy(max_v, max_hbm.at[pl.ds(base, SPB)])

    return pool(x2, lengths, emb_table)


def _mlp_kernel(sum_ref, max_ref, len_ref, w1a_ref, w1b_ref, b1_ref,
                w2_ref, b2_ref, out_ref):
    mean = sum_ref[...] / len_ref[...]
    h = jnp.dot(mean, w1a_ref[...], preferred_element_type=jnp.float32)
    h += jnp.dot(max_ref[...], w1b_ref[...], preferred_element_type=jnp.float32)
    h = jnp.maximum(h + b1_ref[...], 0.0)
    out = jnp.dot(h, w2_ref[...], preferred_element_type=jnp.float32)
    out_ref[...] = out + b2_ref[...]


def _mlp_tc(sum_pooled, max_pooled, lengths_f, W1, b1, W2, b2):
    H = W1.shape[1]
    C = W2.shape[1]
    W1a = W1[:D]
    W1b = W1[D:]
    BLK = 1024
    grid = B // BLK
    return pl.pallas_call(
        _mlp_kernel,
        grid=(grid,),
        in_specs=[
            pl.BlockSpec((BLK, D), lambda i: (i, 0)),
            pl.BlockSpec((BLK, D), lambda i: (i, 0)),
            pl.BlockSpec((BLK, 1), lambda i: (i, 0)),
            pl.BlockSpec((D, H), lambda i: (0, 0)),
            pl.BlockSpec((D, H), lambda i: (0, 0)),
            pl.BlockSpec((1, H), lambda i: (0, 0)),
            pl.BlockSpec((H, C), lambda i: (0, 0)),
            pl.BlockSpec((1, C), lambda i: (0, 0)),
        ],
        out_specs=pl.BlockSpec((BLK, C), lambda i: (i, 0)),
        out_shape=jax.ShapeDtypeStruct((B, C), jnp.float32),
    )(sum_pooled, max_pooled, lengths_f, W1a, W1b, b1[None, :], W2, b2[None, :])


def kernel(x, lengths, emb_table, W1, b1, W2, b2):
    x2 = x.reshape(B, 2, LH).astype(jnp.int32)
    lengths = lengths.astype(jnp.int32)
    sum_pooled, max_pooled = _pool_sc(x2, lengths, emb_table)
    lengths_f = lengths.astype(jnp.float32)[:, None]
    return _mlp_tc(sum_pooled, max_pooled, lengths_f, W1, b1, W2, b2)


# speedup vs baseline: 15.8615x; 1.3181x over previous
import functools

import jax
import jax.numpy as jnp
from jax import lax
from jax.experimental import pallas as pl
from jax.experimental.pallas import tpu as pltpu
from jax.experimental.pallas import tpu_sc as plsc

B, L, V, D = 4096, 200, 100000, 64
NC, NS = 2, 16
NW = NC * NS
SPB = B // NW
LH = L // 2


def _pool_sc(x2, lengths, emb_table):
    mesh = plsc.VectorSubcoreMesh(core_axis_name="c", subcore_axis_name="s")

    @functools.partial(
        pl.kernel,
        out_type=(
            jax.ShapeDtypeStruct((B, D), jnp.float32),
            jax.ShapeDtypeStruct((B, D), jnp.float32),
        ),
        mesh=mesh,
        compiler_params=pltpu.CompilerParams(use_tc_tiling_on_sc=False),
        scratch_types=[
            pltpu.VMEM((SPB, 2, LH), jnp.int32),
            pltpu.VMEM((SPB + 16,), jnp.int32),
            pltpu.VMEM((L, D), jnp.float32),
            pltpu.VMEM((L, D), jnp.float32),
            pltpu.VMEM((L, D), jnp.float32),
            pltpu.VMEM((L, D), jnp.float32),
            pltpu.VMEM((SPB, D), jnp.float32),
            pltpu.VMEM((SPB, D), jnp.float32),
            pltpu.SemaphoreType.DMA,
            pltpu.SemaphoreType.DMA,
            pltpu.SemaphoreType.DMA,
            pltpu.SemaphoreType.DMA,
        ],
    )
    def pool(x_hbm, len_hbm, table_hbm, sum_hbm, max_hbm,
             idx_v, len_v, rows0_v, rows1_v, rows2_v, rows3_v,
             sum_v, max_v, sem0, sem1, sem2, sem3):
        wid = lax.axis_index("s") * NC + lax.axis_index("c")
        base = wid * SPB
        pltpu.sync_copy(x_hbm.at[pl.ds(base, SPB)], idx_v)
        pltpu.sync_copy(len_hbm.at[pl.ds(base, SPB)], len_v.at[pl.ds(0, SPB)])

        neg = jnp.finfo(jnp.float32).min

        def start_gather(b, buf, sem):
            pltpu.async_copy(table_hbm.at[idx_v.at[b, 0]],
                             buf.at[pl.ds(0, LH)], sem)
            pltpu.async_copy(table_hbm.at[idx_v.at[b, 1]],
                             buf.at[pl.ds(LH, LH)], sem)

        def wait_gather(buf, sem):
            pltpu.make_async_copy(table_hbm.at[pl.ds(0, L)], buf, sem).wait()

        def compute(b, rows_v):
            n = len_v[pl.ds(b, 16)][0]
            init = (jnp.zeros((16,), jnp.float32),) * 4 + \
                   (jnp.full((16,), neg, jnp.float32),) * 4

            def row_sm(r, c):
                s0, s1, s2, s3, m0, m1, m2, m3 = c
                v0 = rows_v[r, pl.ds(0, 16)]
                v1 = rows_v[r, pl.ds(16, 16)]
                v2 = rows_v[r, pl.ds(32, 16)]
                v3 = rows_v[r, pl.ds(48, 16)]
                return (s0 + v0, s1 + v1, s2 + v2, s3 + v3,
                        jnp.maximum(m0, v0), jnp.maximum(m1, v1),
                        jnp.maximum(m2, v2), jnp.maximum(m3, v3))

            def row_s(r, c):
                s0, s1, s2, s3, m0, m1, m2, m3 = c
                v0 = rows_v[r, pl.ds(0, 16)]
                v1 = rows_v[r, pl.ds(16, 16)]
                v2 = rows_v[r, pl.ds(32, 16)]
                v3 = rows_v[r, pl.ds(48, 16)]
                return (s0 + v0, s1 + v1, s2 + v2, s3 + v3, m0, m1, m2, m3)

            def quad_sm(i, c):
                r = i * 4
                for j in range(4):
                    c = row_sm(r + j, c)
                return c

            def quad_s(i, c):
                r = up4 + i * 4
                for j in range(4):
                    c = row_s(r + j, c)
                return c

            n4 = n & ~3
            up4 = (n + 3) & ~3
            c = lax.fori_loop(0, n4 // 4, quad_sm, init)
            c = lax.fori_loop(n4, n, row_sm, c)
            c = lax.fori_loop(n, up4, row_s, c)
            s0, s1, s2, s3, m0, m1, m2, m3 = \
                lax.fori_loop(0, (L - up4) // 4, quad_s, c)
            sum_v[b, pl.ds(0, 16)] = s0
            sum_v[b, pl.ds(16, 16)] = s1
            sum_v[b, pl.ds(32, 16)] = s2
            sum_v[b, pl.ds(48, 16)] = s3
            max_v[b, pl.ds(0, 16)] = m0
            max_v[b, pl.ds(16, 16)] = m1
            max_v[b, pl.ds(32, 16)] = m2
            max_v[b, pl.ds(48, 16)] = m3

        bufs = ((rows0_v, sem0), (rows1_v, sem1),
                (rows2_v, sem2), (rows3_v, sem3))
        NBUF = 4
        for k in range(NBUF - 1):
            start_gather(k, *bufs[k])

        @pl.loop(0, SPB, step=NBUF)
        def _quad(i):
            for k in range(NBUF):
                b = i + k
                buf, sem = bufs[k]
                nbuf, nsem = bufs[(k + NBUF - 1) % NBUF]

                @pl.when(b + NBUF - 1 < SPB)
                def _():
                    start_gather(b + NBUF - 1, nbuf, nsem)

                wait_gather(buf, sem)
                compute(b, buf)

        pltpu.sync_copy(sum_v, sum_hbm.at[pl.ds(base, SPB)])
        pltpu.sync_copy(max_v, max_hbm.at[pl.ds(base, SPB)])

    return pool(x2, lengths, emb_table)


def _mlp_kernel(sum_ref, max_ref, len_ref, w1a_ref, w1b_ref, b1_ref,
                w2_ref, b2_ref, out_ref):
    mean = sum_ref[...] / len_ref[...]
    h = jnp.dot(mean, w1a_ref[...], preferred_element_type=jnp.float32)
    h += jnp.dot(max_ref[...], w1b_ref[...], preferred_element_type=jnp.float32)
    h = jnp.maximum(h + b1_ref[...], 0.0)
    out = jnp.dot(h, w2_ref[...], preferred_element_type=jnp.float32)
    out_ref[...] = out + b2_ref[...]


def _mlp_tc(sum_pooled, max_pooled, lengths_f, W1, b1, W2, b2):
    H = W1.shape[1]
    C = W2.shape[1]
    W1a = W1[:D]
    W1b = W1[D:]
    BLK = 1024
    grid = B // BLK
    return pl.pallas_call(
        _mlp_kernel,
        grid=(grid,),
        in_specs=[
            pl.BlockSpec((BLK, D), lambda i: (i, 0)),
            pl.BlockSpec((BLK, D), lambda i: (i, 0)),
            pl.BlockSpec((BLK, 1), lambda i: (i, 0)),
            pl.BlockSpec((D, H), lambda i: (0, 0)),
            pl.BlockSpec((D, H), lambda i: (0, 0)),
            pl.BlockSpec((1, H), lambda i: (0, 0)),
            pl.BlockSpec((H, C), lambda i: (0, 0)),
            pl.BlockSpec((1, C), lambda i: (0, 0)),
        ],
        out_specs=pl.BlockSpec((BLK, C), lambda i: (i, 0)),
        out_shape=jax.ShapeDtypeStruct((B, C), jnp.float32),
    )(sum_pooled, max_pooled, lengths_f, W1a, W1b, b1[None, :], W2, b2[None, :])


def kernel(x, lengths, emb_table, W1, b1, W2, b2):
    x2 = x.reshape(B, 2, LH).astype(jnp.int32)
    lengths = lengths.astype(jnp.int32)
    sum_pooled, max_pooled = _pool_sc(x2, lengths, emb_table)
    lengths_f = lengths.astype(jnp.float32)[:, None]
    return _mlp_tc(sum_pooled, max_pooled, lengths_f, W1, b1, W2, b2)


# speedup vs baseline: 17.1126x; 1.0789x over previous
import functools

import jax
import jax.numpy as jnp
from jax import lax
from jax.experimental import pallas as pl
from jax.experimental.pallas import tpu as pltpu
from jax.experimental.pallas import tpu_sc as plsc

B, L, V, D = 4096, 200, 100000, 64
NC, NS = 2, 16
NW = NC * NS
SPB = B // NW
LH1, LH2 = 128, 72


def _pool_sc(x, lengths, emb_table):
    mesh = plsc.VectorSubcoreMesh(core_axis_name="c", subcore_axis_name="s")

    @functools.partial(
        pl.kernel,
        out_type=(
            jax.ShapeDtypeStruct((B, D), jnp.float32),
            jax.ShapeDtypeStruct((B, D), jnp.float32),
        ),
        mesh=mesh,
        compiler_params=pltpu.CompilerParams(use_tc_tiling_on_sc=False),
        scratch_types=[
            pltpu.VMEM((SPB, L), jnp.int32),
            pltpu.VMEM((SPB + 16,), jnp.int32),
            pltpu.VMEM((L, D), jnp.float32),
            pltpu.VMEM((L, D), jnp.float32),
            pltpu.VMEM((L, D), jnp.float32),
            pltpu.VMEM((L, D), jnp.float32),
            pltpu.VMEM((SPB, D), jnp.float32),
            pltpu.VMEM((SPB, D), jnp.float32),
            pltpu.SemaphoreType.DMA,
            pltpu.SemaphoreType.DMA,
            pltpu.SemaphoreType.DMA,
            pltpu.SemaphoreType.DMA,
        ],
    )
    def pool(x_hbm, len_hbm, table_hbm, sum_hbm, max_hbm,
             idx_v, len_v, rows0_v, rows1_v, rows2_v, rows3_v,
             sum_v, max_v, sem0, sem1, sem2, sem3):
        wid = lax.axis_index("s") * NC + lax.axis_index("c")
        base = wid * SPB
        pltpu.sync_copy(x_hbm.at[pl.ds(base, SPB)], idx_v)
        pltpu.sync_copy(len_hbm.at[pl.ds(base, SPB)], len_v.at[pl.ds(0, SPB)])

        neg = jnp.finfo(jnp.float32).min

        def start_gather(b, buf, sem):
            pltpu.async_copy(table_hbm.at[idx_v.at[b, pl.ds(0, LH1)]],
                             buf.at[pl.ds(0, LH1)], sem)
            pltpu.async_copy(table_hbm.at[idx_v.at[b, pl.ds(LH1, LH2)]],
                             buf.at[pl.ds(LH1, LH2)], sem)

        def wait_gather(buf, sem):
            pltpu.make_async_copy(table_hbm.at[pl.ds(0, L)], buf, sem).wait()

        def compute(b, rows_v):
            n = len_v[pl.ds(b, 16)][0]
            init = (jnp.zeros((16,), jnp.float32),) * 4 + \
                   (jnp.full((16,), neg, jnp.float32),) * 4

            def row_sm(r, c):
                s0, s1, s2, s3, m0, m1, m2, m3 = c
                v0 = rows_v[r, pl.ds(0, 16)]
                v1 = rows_v[r, pl.ds(16, 16)]
                v2 = rows_v[r, pl.ds(32, 16)]
                v3 = rows_v[r, pl.ds(48, 16)]
                return (s0 + v0, s1 + v1, s2 + v2, s3 + v3,
                        jnp.maximum(m0, v0), jnp.maximum(m1, v1),
                        jnp.maximum(m2, v2), jnp.maximum(m3, v3))

            def row_s(r, c):
                s0, s1, s2, s3, m0, m1, m2, m3 = c
                v0 = rows_v[r, pl.ds(0, 16)]
                v1 = rows_v[r, pl.ds(16, 16)]
                v2 = rows_v[r, pl.ds(32, 16)]
                v3 = rows_v[r, pl.ds(48, 16)]
                return (s0 + v0, s1 + v1, s2 + v2, s3 + v3, m0, m1, m2, m3)

            def quad_sm(i, c):
                r = i * 4
                for j in range(4):
                    c = row_sm(r + j, c)
                return c

            def quad_s(i, c):
                r = up4 + i * 4
                for j in range(4):
                    c = row_s(r + j, c)
                return c

            n4 = n & ~3
            up4 = (n + 3) & ~3
            c = lax.fori_loop(0, n4 // 4, quad_sm, init)
            c = lax.fori_loop(n4, n, row_sm, c)
            c = lax.fori_loop(n, up4, row_s, c)
            s0, s1, s2, s3, m0, m1, m2, m3 = \
                lax.fori_loop(0, (L - up4) // 4, quad_s, c)
            sum_v[b, pl.ds(0, 16)] = s0
            sum_v[b, pl.ds(16, 16)] = s1
            sum_v[b, pl.ds(32, 16)] = s2
            sum_v[b, pl.ds(48, 16)] = s3
            max_v[b, pl.ds(0, 16)] = m0
            max_v[b, pl.ds(16, 16)] = m1
            max_v[b, pl.ds(32, 16)] = m2
            max_v[b, pl.ds(48, 16)] = m3

        bufs = ((rows0_v, sem0), (rows1_v, sem1),
                (rows2_v, sem2), (rows3_v, sem3))
        NBUF = 4
        for k in range(NBUF - 1):
            start_gather(k, *bufs[k])

        @pl.loop(0, SPB, step=NBUF)
        def _quad(i):
            for k in range(NBUF):
                b = i + k
                buf, sem = bufs[k]
                nbuf, nsem = bufs[(k + NBUF - 1) % NBUF]

                @pl.when(b + NBUF - 1 < SPB)
                def _():
                    start_gather(b + NBUF - 1, nbuf, nsem)

                wait_gather(buf, sem)
                compute(b, buf)

        pltpu.sync_copy(sum_v, sum_hbm.at[pl.ds(base, SPB)])
        pltpu.sync_copy(max_v, max_hbm.at[pl.ds(base, SPB)])

    return pool(x, lengths, emb_table)


def _mlp_kernel(sum_ref, max_ref, len_ref, w1a_ref, w1b_ref, b1_ref,
                w2_ref, b2_ref, out_ref):
    mean = sum_ref[...] / len_ref[...]
    h = jnp.dot(mean, w1a_ref[...], preferred_element_type=jnp.float32)
    h += jnp.dot(max_ref[...], w1b_ref[...], preferred_element_type=jnp.float32)
    h = jnp.maximum(h + b1_ref[...], 0.0)
    out = jnp.dot(h, w2_ref[...], preferred_element_type=jnp.float32)
    out_ref[...] = out + b2_ref[...]


def _mlp_tc(sum_pooled, max_pooled, lengths_f, W1, b1, W2, b2):
    H = W1.shape[1]
    C = W2.shape[1]
    W1a = W1[:D]
    W1b = W1[D:]
    BLK = 1024
    grid = B // BLK
    return pl.pallas_call(
        _mlp_kernel,
        grid=(grid,),
        in_specs=[
            pl.BlockSpec((BLK, D), lambda i: (i, 0)),
            pl.BlockSpec((BLK, D), lambda i: (i, 0)),
            pl.BlockSpec((BLK, 1), lambda i: (i, 0)),
            pl.BlockSpec((D, H), lambda i: (0, 0)),
            pl.BlockSpec((D, H), lambda i: (0, 0)),
            pl.BlockSpec((1, H), lambda i: (0, 0)),
            pl.BlockSpec((H, C), lambda i: (0, 0)),
            pl.BlockSpec((1, C), lambda i: (0, 0)),
        ],
        out_specs=pl.BlockSpec((BLK, C), lambda i: (i, 0)),
        out_shape=jax.ShapeDtypeStruct((B, C), jnp.float32),
    )(sum_pooled, max_pooled, lengths_f, W1a, W1b, b1[None, :], W2, b2[None, :])


def kernel(x, lengths, emb_table, W1, b1, W2, b2):
    sum_pooled, max_pooled = _pool_sc(x, lengths, emb_table)
    lengths_f = lengths.astype(jnp.float32)[:, None]
    return _mlp_tc(sum_pooled, max_pooled, lengths_f, W1, b1, W2, b2)


# speedup vs baseline: 17.8236x; 1.0416x over previous
import functools

import jax
import jax.numpy as jnp
from jax import lax
from jax.experimental import pallas as pl
from jax.experimental.pallas import tpu as pltpu
from jax.experimental.pallas import tpu_sc as plsc

B, L, V, D = 4096, 200, 100000, 64
NC, NS = 2, 16
NW = NC * NS
SPB = B // NW
LH1, LH2 = 128, 72


def _pool_sc(x, lengths, emb_table):
    mesh = plsc.VectorSubcoreMesh(core_axis_name="c", subcore_axis_name="s")

    @functools.partial(
        pl.kernel,
        out_type=jax.ShapeDtypeStruct((B, 2 * D), jnp.float32),
        mesh=mesh,
        compiler_params=pltpu.CompilerParams(use_tc_tiling_on_sc=False),
        scratch_types=[
            pltpu.VMEM((SPB, L), jnp.int32),
            pltpu.VMEM((SPB + 16,), jnp.int32),
            pltpu.VMEM((L, D), jnp.float32),
            pltpu.VMEM((L, D), jnp.float32),
            pltpu.VMEM((L, D), jnp.float32),
            pltpu.VMEM((L, D), jnp.float32),
            pltpu.VMEM((SPB, 2 * D), jnp.float32),
            pltpu.SemaphoreType.DMA,
            pltpu.SemaphoreType.DMA,
            pltpu.SemaphoreType.DMA,
            pltpu.SemaphoreType.DMA,
        ],
    )
    def pool(x_hbm, len_hbm, table_hbm, rep_hbm,
             idx_v, len_v, rows0_v, rows1_v, rows2_v, rows3_v,
             rep_v, sem0, sem1, sem2, sem3):
        wid = lax.axis_index("s") * NC + lax.axis_index("c")
        base = wid * SPB
        pltpu.sync_copy(x_hbm.at[pl.ds(base, SPB)], idx_v)
        pltpu.sync_copy(len_hbm.at[pl.ds(base, SPB)], len_v.at[pl.ds(0, SPB)])

        neg = jnp.finfo(jnp.float32).min

        def start_gather(b, buf, sem):
            pltpu.async_copy(table_hbm.at[idx_v.at[b, pl.ds(0, LH1)]],
                             buf.at[pl.ds(0, LH1)], sem)
            pltpu.async_copy(table_hbm.at[idx_v.at[b, pl.ds(LH1, LH2)]],
                             buf.at[pl.ds(LH1, LH2)], sem)

        def wait_gather(buf, sem):
            pltpu.make_async_copy(table_hbm.at[pl.ds(0, L)], buf, sem).wait()

        def compute(b, rows_v):
            n = len_v[pl.ds(b, 16)][0]
            init = (jnp.zeros((16,), jnp.float32),) * 4 + \
                   (jnp.full((16,), neg, jnp.float32),) * 4

            def row_sm(r, c):
                s0, s1, s2, s3, m0, m1, m2, m3 = c
                v0 = rows_v[r, pl.ds(0, 16)]
                v1 = rows_v[r, pl.ds(16, 16)]
                v2 = rows_v[r, pl.ds(32, 16)]
                v3 = rows_v[r, pl.ds(48, 16)]
                return (s0 + v0, s1 + v1, s2 + v2, s3 + v3,
                        jnp.maximum(m0, v0), jnp.maximum(m1, v1),
                        jnp.maximum(m2, v2), jnp.maximum(m3, v3))

            def row_s(r, c):
                s0, s1, s2, s3, m0, m1, m2, m3 = c
                v0 = rows_v[r, pl.ds(0, 16)]
                v1 = rows_v[r, pl.ds(16, 16)]
                v2 = rows_v[r, pl.ds(32, 16)]
                v3 = rows_v[r, pl.ds(48, 16)]
                return (s0 + v0, s1 + v1, s2 + v2, s3 + v3, m0, m1, m2, m3)

            def oct_sm(i, c):
                r = i * 8
                for j in range(8):
                    c = row_sm(r + j, c)
                return c

            def oct_s(i, c):
                r = up8 + i * 8
                for j in range(8):
                    c = row_s(r + j, c)
                return c

            n8 = n & ~7
            up8 = (n + 7) & ~7
            c = lax.fori_loop(0, n8 // 8, oct_sm, init)
            c = lax.fori_loop(n8, n, row_sm, c)
            c = lax.fori_loop(n, up8, row_s, c)
            s0, s1, s2, s3, m0, m1, m2, m3 = \
                lax.fori_loop(0, (L - up8) // 8, oct_s, c)
            rep_v[b, pl.ds(0, 16)] = s0
            rep_v[b, pl.ds(16, 16)] = s1
            rep_v[b, pl.ds(32, 16)] = s2
            rep_v[b, pl.ds(48, 16)] = s3
            rep_v[b, pl.ds(64, 16)] = m0
            rep_v[b, pl.ds(80, 16)] = m1
            rep_v[b, pl.ds(96, 16)] = m2
            rep_v[b, pl.ds(112, 16)] = m3

        bufs = ((rows0_v, sem0), (rows1_v, sem1),
                (rows2_v, sem2), (rows3_v, sem3))
        NBUF = 4
        for k in range(NBUF - 1):
            start_gather(k, *bufs[k])

        @pl.loop(0, SPB, step=NBUF)
        def _quad(i):
            for k in range(NBUF):
                b = i + k
                buf, sem = bufs[k]
                nbuf, nsem = bufs[(k + NBUF - 1) % NBUF]

                @pl.when(b + NBUF - 1 < SPB)
                def _():
                    start_gather(b + NBUF - 1, nbuf, nsem)

                wait_gather(buf, sem)
                compute(b, buf)

        pltpu.sync_copy(rep_v, rep_hbm.at[pl.ds(base, SPB)])

    return pool(x, lengths, emb_table)


def _mlp_kernel(rep_ref, len_ref, w1_ref, b1_ref, w2_ref, b2_ref, out_ref):
    rep = rep_ref[...]
    mean = rep[:, :D] / len_ref[...]
    rep2 = jnp.concatenate([mean, rep[:, D:]], axis=1)
    h = jnp.dot(rep2, w1_ref[...], preferred_element_type=jnp.float32)
    h = jnp.maximum(h + b1_ref[...], 0.0)
    out = jnp.dot(h, w2_ref[...], preferred_element_type=jnp.float32)
    out_ref[...] = out + b2_ref[...]


def _mlp_tc(rep_raw, lengths_f, W1, b1, W2, b2):
    H = W1.shape[1]
    C = W2.shape[1]
    BLK = 1024
    grid = B // BLK
    return pl.pallas_call(
        _mlp_kernel,
        grid=(grid,),
        in_specs=[
            pl.BlockSpec((BLK, 2 * D), lambda i: (i, 0)),
            pl.BlockSpec((BLK, 1), lambda i: (i, 0)),
            pl.BlockSpec((2 * D, H), lambda i: (0, 0)),
            pl.BlockSpec((1, H), lambda i: (0, 0)),
            pl.BlockSpec((H, C), lambda i: (0, 0)),
            pl.BlockSpec((1, C), lambda i: (0, 0)),
        ],
        out_specs=pl.BlockSpec((BLK, C), lambda i: (i, 0)),
        out_shape=jax.ShapeDtypeStruct((B, C), jnp.float32),
    )(rep_raw, lengths_f, W1, b1[None, :], W2, b2[None, :])


def kernel(x, lengths, emb_table, W1, b1, W2, b2):
    rep_raw = _pool_sc(x, lengths, emb_table)
    lengths_f = lengths.astype(jnp.float32)[:, None]
    return _mlp_tc(rep_raw, lengths_f, W1, b1, W2, b2)
